# baseline (speedup 1.0000x reference)
import functools

import jax
import jax.numpy as jnp
from jax import lax
from jax.experimental import pallas as pl
from jax.experimental.pallas import tpu as pltpu
from jax.experimental.pallas import tpu_sc as plsc

_NC = 2
_NS = 16
_NW = _NC * _NS
_IDX_CHUNK = 128


def _encoder_body(x_ref, w_ref, b_ref, o_ref):
    o_ref[...] = jax.nn.relu(
        jnp.dot(x_ref[...], w_ref[...], preferred_element_type=jnp.float32)
        + b_ref[...]
    )


def _node_encoder(x, w, b):
    n, _ = x.shape
    h = w.shape[1]
    return pl.pallas_call(
        _encoder_body,
        out_shape=jax.ShapeDtypeStruct((n, h), jnp.float32),
    )(x, w, b.reshape(1, h))


def _edge_layer_body(e_true, tile, hdim, eit_ref, wet_ref, be_ref, wit_ref,
                     bi_ref, wot_ref, bo_ref, hd_ref, hs_ref, mi_ref, mo_ref):
    t = pl.program_id(0)
    eht = jax.nn.relu(
        jnp.dot(wet_ref[...], eit_ref[...], preferred_element_type=jnp.float32)
        + be_ref[...]
    )
    cols = t * tile + lax.broadcasted_iota(jnp.int32, (1, tile), 1)
    valid = cols < e_true

    def msgs(wt_ref, b_ref, hx_ref, out_ref):
        tit = jax.nn.relu(
            jnp.dot(wt_ref[...], eht, preferred_element_type=jnp.float32)
            + b_ref[...]
        )
        hxt = hx_ref[...].T
        p = tit.reshape(-1, hdim, tile) * hxt[None, :, :]
        m = p.sum(axis=1)
        out_ref[...] = jnp.where(valid, m, 0.0).T

    msgs(wit_ref, bi_ref, hd_ref, mi_ref)
    msgs(wot_ref, bo_ref, hs_ref, mo_ref)


def _edge_layer(e_true, edge_inp_t_pad, w_edge, b_edge, w_mi, b_mi, w_mo,
                b_mo, hd, hs, tile=1024):
    e_pad = edge_inp_t_pad.shape[1]
    h = w_edge.shape[1]
    h2 = w_mi.shape[1]
    grid = (e_pad // tile,)
    const = lambda i: (0, 0)
    return pl.pallas_call(
        functools.partial(_edge_layer_body, e_true, tile, h),
        grid=grid,
        in_specs=[
            pl.BlockSpec((2, tile), lambda i: (0, i)),
            pl.BlockSpec((h, 2), const),
            pl.BlockSpec((h, 1), const),
            pl.BlockSpec((h2, h), const),
            pl.BlockSpec((h2, 1), const),
            pl.BlockSpec((h2, h), const),
            pl.BlockSpec((h2, 1), const),
            pl.BlockSpec((tile, h), lambda i: (i, 0)),
            pl.BlockSpec((tile, h), lambda i: (i, 0)),
        ],
        out_specs=[
            pl.BlockSpec((tile, h // 2), lambda i: (i, 0)),
            pl.BlockSpec((tile, h // 2), lambda i: (i, 0)),
        ],
        out_shape=[
            jax.ShapeDtypeStruct((e_pad, h // 2), jnp.float32),
            jax.ShapeDtypeStruct((e_pad, h // 2), jnp.float32),
        ],
    )(edge_inp_t_pad, w_edge.T, b_edge.reshape(h, 1), w_mi.T,
      b_mi.reshape(h2, 1), w_mo.T, b_mo.reshape(h2, 1), hd, hs)


def _update_body(h_ref, mi_ref, mo_ref, o_ref):
    mi = mi_ref[0] + mi_ref[1]
    mo = mo_ref[0] + mo_ref[1]
    o_ref[...] = jax.nn.relu(h_ref[...] + jnp.concatenate([mi, mo], axis=1))


def _node_update(h, mi_p, mo_p):
    return pl.pallas_call(
        _update_body,
        out_shape=jax.ShapeDtypeStruct(h.shape, jnp.float32),
    )(h, mi_p, mo_p)


def _readout_body(h_ref, mi_ref, mo_ref, wg_ref, bg_ref, wf_ref, bf_ref,
                  w1_ref, b1_ref, w2_ref, b2_ref, o_ref):
    mi = mi_ref[0] + mi_ref[1]
    mo = mo_ref[0] + mo_ref[1]
    h = jax.nn.relu(h_ref[...] + jnp.concatenate([mi, mo], axis=1))
    g = jnp.dot(h, wg_ref[...], preferred_element_type=jnp.float32) + bg_ref[...]
    p = jnp.exp(g - jnp.max(g))
    z = jnp.sum(p)
    feat = jnp.dot(h, wf_ref[...], preferred_element_type=jnp.float32) + bf_ref[...]
    r = jnp.sum(p * feat, axis=0, keepdims=True) / z
    o1 = jax.nn.relu(
        jnp.dot(r, w1_ref[...], preferred_element_type=jnp.float32) + b1_ref[...]
    )
    o_ref[...] = jnp.dot(o1, w2_ref[...], preferred_element_type=jnp.float32) + b2_ref[...]


def _readout(h, mi_p, mo_p, w_gate, b_gate, w_feat, b_feat, w_f1, b_f1, w_f2, b_f2):
    hdim = h.shape[1]
    out = pl.pallas_call(
        _readout_body,
        out_shape=jax.ShapeDtypeStruct((1, 1), jnp.float32),
    )(h, mi_p, mo_p, w_gate, b_gate.reshape(1, 1), w_feat,
      b_feat.reshape(1, hdim), w_f1, b_f1.reshape(1, hdim), w_f2,
      b_f2.reshape(1, 1))
    return out.reshape(())


def _gather_body(chunk, nidx, h_hbm, dst_hbm, src_hbm, hd_hbm, hs_hbm,
                 idx_v, rows_v, sem):
    c = lax.axis_index("c")
    s = lax.axis_index("s")
    wid = c * _NS + s
    base = wid * chunk

    def gather_one(idx3, out_hbm):
        pltpu.sync_copy(idx3.at[wid], idx_v)
        descs = []
        for j in range(nidx):
            descs.append(pltpu.async_copy(
                h_hbm.at[idx_v.at[j]],
                rows_v.at[pl.ds(j * _IDX_CHUNK, _IDX_CHUNK)],
                sem,
            ))
        for d in descs:
            d.wait()
        pltpu.sync_copy(rows_v, out_hbm.at[pl.ds(base, chunk)])

    gather_one(dst_hbm, hd_hbm)
    gather_one(src_hbm, hs_hbm)


def _sc_gather(h, dst3, src3):
    n, hdim = h.shape
    nw, nidx, _ = dst3.shape
    chunk = nidx * _IDX_CHUNK
    e_pad = nw * chunk
    mesh = plsc.VectorSubcoreMesh(core_axis_name="c", subcore_axis_name="s")
    kfn = pl.kernel(
        functools.partial(_gather_body, chunk, nidx),
        out_type=[
            jax.ShapeDtypeStruct((e_pad, hdim), jnp.float32),
            jax.ShapeDtypeStruct((e_pad, hdim), jnp.float32),
        ],
        mesh=mesh,
        scratch_types=[
            pltpu.VMEM((nidx, _IDX_CHUNK), jnp.int32),
            pltpu.VMEM((chunk, hdim), jnp.float32),
            pltpu.SemaphoreType.DMA,
        ],
        compiler_params=pltpu.CompilerParams(use_tc_tiling_on_sc=False),
    )
    return kfn(h, dst3, src3)


def _scatter_body(nidx, mi3_hbm, mo3_hbm, dst_hbm, src_hbm, zero_hbm,
                  mip_hbm, mop_hbm, msg_v, idx_v, acc_i, acc_o):
    c = lax.axis_index("c")
    s = lax.axis_index("s")
    wid = c * _NS + s

    @pl.when(s == 0)
    def _():
        pltpu.sync_copy(zero_hbm, acc_i)

    @pl.when(s == 1)
    def _():
        pltpu.sync_copy(zero_hbm, acc_o)

    plsc.subcore_barrier()

    def scatter_one(msg3, idx3, acc):
        pltpu.sync_copy(msg3.at[wid], msg_v)
        pltpu.sync_copy(idx3.at[wid], idx_v)
        for j in range(nidx):
            pltpu.sync_copy(
                msg_v.at[pl.ds(j * _IDX_CHUNK, _IDX_CHUNK)],
                acc.at[idx_v.at[j]],
                add=True,
            )

    scatter_one(mi3_hbm, dst_hbm, acc_i)
    scatter_one(mo3_hbm, src_hbm, acc_o)

    plsc.subcore_barrier()

    @pl.when(s == 0)
    def _():
        pltpu.sync_copy(acc_i, mip_hbm.at[c])

    @pl.when(s == 1)
    def _():
        pltpu.sync_copy(acc_o, mop_hbm.at[c])


def _sc_scatter(mi3, mo3, dst3, src3, zeros_nk):
    nw, chunk, k = mi3.shape
    nidx = chunk // _IDX_CHUNK
    n = zeros_nk.shape[0]
    mesh = plsc.VectorSubcoreMesh(core_axis_name="c", subcore_axis_name="s")
    kfn = pl.kernel(
        functools.partial(_scatter_body, nidx),
        out_type=[
            jax.ShapeDtypeStruct((_NC, n, k), jnp.float32),
            jax.ShapeDtypeStruct((_NC, n, k), jnp.float32),
        ],
        mesh=mesh,
        scratch_types=[
            pltpu.VMEM((chunk, k), jnp.float32),
            pltpu.VMEM((nidx, _IDX_CHUNK), jnp.int32),
            pltpu.VMEM_SHARED((n, k), jnp.float32),
            pltpu.VMEM_SHARED((n, k), jnp.float32),
        ],
        compiler_params=pltpu.CompilerParams(use_tc_tiling_on_sc=False),
    )
    return kfn(mi3, mo3, dst3, src3, zeros_nk)


def kernel(node_inp, edge_inp, W_node, b_node, W_edge, b_edge, W_mi, b_mi,
           W_mo, b_mo, W_gate, b_gate, W_feat, b_feat, W_f1, b_f1, W_f2,
           b_f2, edge_index):
    n = node_inp.shape[0]
    e = edge_inp.shape[0]
    hdim = W_node.shape[1]
    nlayers = W_mi.shape[0]

    unit = _NW * _IDX_CHUNK
    e_pad = ((e + unit - 1) // unit) * unit
    chunk = e_pad // _NW
    nidx = chunk // _IDX_CHUNK

    pad = e_pad - e
    src = jnp.pad(edge_index[0], (0, pad))
    dst = jnp.pad(edge_index[1], (0, pad))
    dst3 = dst.reshape(_NW, nidx, _IDX_CHUNK)
    src3 = src.reshape(_NW, nidx, _IDX_CHUNK)
    edge_inp_t_pad = jnp.pad(edge_inp, ((0, pad), (0, 0))).T
    zeros_nk = jnp.zeros((n, hdim // 2), jnp.float32)

    h = _node_encoder(node_inp, W_node, b_node)

    mi_p = mo_p = None
    for l in range(nlayers):
        if l > 0:
            h = _node_update(h, mi_p, mo_p)
        hd, hs = h[dst], h[src]
        mi_msg, mo_msg = _edge_layer(
            e, edge_inp_t_pad, W_edge, b_edge, W_mi[l], b_mi[l], W_mo[l],
            b_mo[l], hd, hs)
        mi_full = jax.ops.segment_sum(mi_msg, dst, num_segments=n)
        mo_full = jax.ops.segment_sum(mo_msg, src, num_segments=n)
        mi_p = jnp.stack([mi_full, zeros_nk])
        mo_p = jnp.stack([mo_full, zeros_nk])

    return _readout(h, mi_p, mo_p, W_gate, b_gate, W_feat, b_feat,
                    W_f1, b_f1, W_f2, b_f2)


# speedup vs baseline: 2.1531x; 2.1531x over previous
import functools

import jax
import jax.numpy as jnp
from jax import lax
from jax.experimental import pallas as pl
from jax.experimental.pallas import tpu as pltpu
from jax.experimental.pallas import tpu_sc as plsc

_NC = 2
_NS = 16
_NW = _NC * _NS
_IDX_CHUNK = 128


def _encoder_body(x_ref, w_ref, b_ref, o_ref):
    o_ref[...] = jax.nn.relu(
        jnp.dot(x_ref[...], w_ref[...], preferred_element_type=jnp.float32)
        + b_ref[...]
    )


def _node_encoder(x, w, b):
    n, _ = x.shape
    h = w.shape[1]
    return pl.pallas_call(
        _encoder_body,
        out_shape=jax.ShapeDtypeStruct((n, h), jnp.float32),
    )(x, w, b.reshape(1, h))


def _edge_layer_body(e_true, tile, hdim, eit_ref, wet_ref, be_ref, wit_ref,
                     bi_ref, wot_ref, bo_ref, hd_ref, hs_ref, mi_ref, mo_ref):
    t = pl.program_id(0)
    eht = jax.nn.relu(
        jnp.dot(wet_ref[...], eit_ref[...], preferred_element_type=jnp.float32)
        + be_ref[...]
    )
    cols = t * tile + lax.broadcasted_iota(jnp.int32, (1, tile), 1)
    valid = cols < e_true

    def msgs(wt_ref, b_ref, hx_ref, out_ref):
        tit = jax.nn.relu(
            jnp.dot(wt_ref[...], eht, preferred_element_type=jnp.float32)
            + b_ref[...]
        )
        hxt = hx_ref[...].T
        p = tit.reshape(-1, hdim, tile) * hxt[None, :, :]
        m = p.sum(axis=1)
        out_ref[...] = jnp.where(valid, m, 0.0).T

    msgs(wit_ref, bi_ref, hd_ref, mi_ref)
    msgs(wot_ref, bo_ref, hs_ref, mo_ref)


def _edge_layer(e_true, edge_inp_t_pad, w_edge, b_edge, w_mi, b_mi, w_mo,
                b_mo, hd, hs, tile=2048):
    e_pad = edge_inp_t_pad.shape[1]
    h = w_edge.shape[1]
    h2 = w_mi.shape[1]
    grid = (e_pad // tile,)
    const = lambda i: (0, 0)
    return pl.pallas_call(
        functools.partial(_edge_layer_body, e_true, tile, h),
        grid=grid,
        in_specs=[
            pl.BlockSpec((2, tile), lambda i: (0, i)),
            pl.BlockSpec((h, 2), const),
            pl.BlockSpec((h, 1), const),
            pl.BlockSpec((h2, h), const),
            pl.BlockSpec((h2, 1), const),
            pl.BlockSpec((h2, h), const),
            pl.BlockSpec((h2, 1), const),
            pl.BlockSpec((tile, h), lambda i: (i, 0)),
            pl.BlockSpec((tile, h), lambda i: (i, 0)),
        ],
        out_specs=[
            pl.BlockSpec((tile, h // 2), lambda i: (i, 0)),
            pl.BlockSpec((tile, h // 2), lambda i: (i, 0)),
        ],
        out_shape=[
            jax.ShapeDtypeStruct((e_pad, h // 2), jnp.float32),
            jax.ShapeDtypeStruct((e_pad, h // 2), jnp.float32),
        ],
    )(edge_inp_t_pad, w_edge.T, b_edge.reshape(h, 1), w_mi.T,
      b_mi.reshape(h2, 1), w_mo.T, b_mo.reshape(h2, 1), hd, hs)


def _update_body(h_ref, mi_ref, mo_ref, o_ref):
    mi = mi_ref[0] + mi_ref[1]
    mo = mo_ref[0] + mo_ref[1]
    o_ref[...] = jax.nn.relu(h_ref[...] + jnp.concatenate([mi, mo], axis=1))


def _node_update(h, mi_p, mo_p):
    return pl.pallas_call(
        _update_body,
        out_shape=jax.ShapeDtypeStruct(h.shape, jnp.float32),
    )(h, mi_p, mo_p)


def _readout_body(h_ref, mi_ref, mo_ref, wg_ref, bg_ref, wf_ref, bf_ref,
                  w1_ref, b1_ref, w2_ref, b2_ref, o_ref):
    mi = mi_ref[0] + mi_ref[1]
    mo = mo_ref[0] + mo_ref[1]
    h = jax.nn.relu(h_ref[...] + jnp.concatenate([mi, mo], axis=1))
    g = jnp.dot(h, wg_ref[...], preferred_element_type=jnp.float32) + bg_ref[...]
    p = jnp.exp(g - jnp.max(g))
    z = jnp.sum(p)
    feat = jnp.dot(h, wf_ref[...], preferred_element_type=jnp.float32) + bf_ref[...]
    r = jnp.sum(p * feat, axis=0, keepdims=True) / z
    o1 = jax.nn.relu(
        jnp.dot(r, w1_ref[...], preferred_element_type=jnp.float32) + b1_ref[...]
    )
    o_ref[...] = jnp.dot(o1, w2_ref[...], preferred_element_type=jnp.float32) + b2_ref[...]


def _readout(h, mi_p, mo_p, w_gate, b_gate, w_feat, b_feat, w_f1, b_f1, w_f2, b_f2):
    hdim = h.shape[1]
    out = pl.pallas_call(
        _readout_body,
        out_shape=jax.ShapeDtypeStruct((1, 1), jnp.float32),
    )(h, mi_p, mo_p, w_gate, b_gate.reshape(1, 1), w_feat,
      b_feat.reshape(1, hdim), w_f1, b_f1.reshape(1, hdim), w_f2,
      b_f2.reshape(1, 1))
    return out.reshape(())


def _gather_body(chunk, nidx, h_hbm, dst_hbm, src_hbm, hd_hbm, hs_hbm,
                 idx_v, rows_v, sem):
    c = lax.axis_index("c")
    s = lax.axis_index("s")
    wid = c * _NS + s
    base = wid * chunk

    def gather_one(idx3, out_hbm):
        pltpu.sync_copy(idx3.at[wid], idx_v)
        descs = []
        for j in range(nidx):
            descs.append(pltpu.async_copy(
                h_hbm.at[idx_v.at[j]],
                rows_v.at[pl.ds(j * _IDX_CHUNK, _IDX_CHUNK)],
                sem,
            ))
        for d in descs:
            d.wait()
        pltpu.sync_copy(rows_v, out_hbm.at[pl.ds(base, chunk)])

    gather_one(dst_hbm, hd_hbm)
    gather_one(src_hbm, hs_hbm)


def _sc_gather(h, dst3, src3):
    n, hdim = h.shape
    nw, nidx, _ = dst3.shape
    chunk = nidx * _IDX_CHUNK
    e_pad = nw * chunk
    mesh = plsc.VectorSubcoreMesh(core_axis_name="c", subcore_axis_name="s")
    kfn = pl.kernel(
        functools.partial(_gather_body, chunk, nidx),
        out_type=[
            jax.ShapeDtypeStruct((e_pad, hdim), jnp.float32),
            jax.ShapeDtypeStruct((e_pad, hdim), jnp.float32),
        ],
        mesh=mesh,
        scratch_types=[
            pltpu.VMEM((nidx, _IDX_CHUNK), jnp.int32),
            pltpu.VMEM((chunk, hdim), jnp.float32),
            pltpu.SemaphoreType.DMA,
        ],
        compiler_params=pltpu.CompilerParams(use_tc_tiling_on_sc=False),
    )
    return kfn(h, dst3, src3)


def _scatter_body(nidx, mi3_hbm, mo3_hbm, dst_hbm, src_hbm, zero_hbm,
                  mip_hbm, mop_hbm, msg_v, idx_v, acc_i, acc_o):
    c = lax.axis_index("c")
    s = lax.axis_index("s")
    wid = c * _NS + s

    @pl.when(s == 0)
    def _():
        pltpu.sync_copy(zero_hbm, acc_i)

    @pl.when(s == 1)
    def _():
        pltpu.sync_copy(zero_hbm, acc_o)

    plsc.subcore_barrier()

    def scatter_one(msg3, idx3, acc):
        pltpu.sync_copy(msg3.at[wid], msg_v)
        pltpu.sync_copy(idx3.at[wid], idx_v)
        for j in range(nidx):
            pltpu.sync_copy(
                msg_v.at[pl.ds(j * _IDX_CHUNK, _IDX_CHUNK)],
                acc.at[idx_v.at[j]],
                add=True,
            )

    scatter_one(mi3_hbm, dst_hbm, acc_i)
    scatter_one(mo3_hbm, src_hbm, acc_o)

    plsc.subcore_barrier()

    @pl.when(s == 0)
    def _():
        pltpu.sync_copy(acc_i, mip_hbm.at[c])

    @pl.when(s == 1)
    def _():
        pltpu.sync_copy(acc_o, mop_hbm.at[c])


def _sc_scatter(mi3, mo3, dst3, src3, zeros_nk):
    nw, chunk, k = mi3.shape
    nidx = chunk // _IDX_CHUNK
    n = zeros_nk.shape[0]
    mesh = plsc.VectorSubcoreMesh(core_axis_name="c", subcore_axis_name="s")
    kfn = pl.kernel(
        functools.partial(_scatter_body, nidx),
        out_type=[
            jax.ShapeDtypeStruct((_NC, n, k), jnp.float32),
            jax.ShapeDtypeStruct((_NC, n, k), jnp.float32),
        ],
        mesh=mesh,
        scratch_types=[
            pltpu.VMEM((chunk, k), jnp.float32),
            pltpu.VMEM((nidx, _IDX_CHUNK), jnp.int32),
            pltpu.VMEM_SHARED((n, k), jnp.float32),
            pltpu.VMEM_SHARED((n, k), jnp.float32),
        ],
        compiler_params=pltpu.CompilerParams(use_tc_tiling_on_sc=False),
    )
    return kfn(mi3, mo3, dst3, src3, zeros_nk)


def kernel(node_inp, edge_inp, W_node, b_node, W_edge, b_edge, W_mi, b_mi,
           W_mo, b_mo, W_gate, b_gate, W_feat, b_feat, W_f1, b_f1, W_f2,
           b_f2, edge_index):
    n = node_inp.shape[0]
    e = edge_inp.shape[0]
    hdim = W_node.shape[1]
    nlayers = W_mi.shape[0]

    unit = _NW * _IDX_CHUNK
    e_pad = ((e + unit - 1) // unit) * unit
    chunk = e_pad // _NW
    nidx = chunk // _IDX_CHUNK

    pad = e_pad - e
    src = jnp.pad(edge_index[0], (0, pad))
    dst = jnp.pad(edge_index[1], (0, pad))
    dst3 = dst.reshape(_NW, nidx, _IDX_CHUNK)
    src3 = src.reshape(_NW, nidx, _IDX_CHUNK)
    edge_inp_t_pad = jnp.pad(edge_inp, ((0, pad), (0, 0))).T
    zeros_nk = jnp.zeros((n, hdim // 2), jnp.float32)

    h = _node_encoder(node_inp, W_node, b_node)

    mi_p = mo_p = None
    for l in range(nlayers):
        if l > 0:
            h = _node_update(h, mi_p, mo_p)
        hd, hs = _sc_gather(h, dst3, src3)
        mi_msg, mo_msg = _edge_layer(
            e, edge_inp_t_pad, W_edge, b_edge, W_mi[l], b_mi[l], W_mo[l],
            b_mo[l], hd, hs)
        mi_p, mo_p = _sc_scatter(
            mi_msg.reshape(_NW, chunk, hdim // 2),
            mo_msg.reshape(_NW, chunk, hdim // 2),
            dst3, src3, zeros_nk)

    return _readout(h, mi_p, mo_p, W_gate, b_gate, W_feat, b_feat,
                    W_f1, b_f1, W_f2, b_f2)


# speedup vs baseline: 2.3838x; 1.1072x over previous
import functools

import jax
import jax.numpy as jnp
from jax import lax
from jax.experimental import pallas as pl
from jax.experimental.pallas import tpu as pltpu
from jax.experimental.pallas import tpu_sc as plsc

_NC = 2
_NS = 16
_NW = _NC * _NS
_IDX_CHUNK = 128


def _encoder_body(x_ref, w_ref, b_ref, o_ref):
    o_ref[...] = jax.nn.relu(
        jnp.dot(x_ref[...], w_ref[...], preferred_element_type=jnp.float32)
        + b_ref[...]
    )


def _node_encoder(x, w, b):
    n, _ = x.shape
    h = w.shape[1]
    return pl.pallas_call(
        _encoder_body,
        out_shape=jax.ShapeDtypeStruct((n, h), jnp.float32),
    )(x, w, b.reshape(1, h))


def _edge_layer_body(e_true, base, tile, hdim, eit_ref, wet_ref, be_ref,
                     wit_ref, bi_ref, wot_ref, bo_ref, hd_ref, hs_ref,
                     mi_ref, mo_ref):
    t = pl.program_id(0)
    eht = jax.nn.relu(
        jnp.dot(wet_ref[...], eit_ref[...], preferred_element_type=jnp.float32)
        + be_ref[...]
    )
    cols = base + t * tile + lax.broadcasted_iota(jnp.int32, (1, tile), 1)
    valid = cols < e_true

    def msgs(wt_ref, b_ref, hx_ref, out_ref):
        tit = jax.nn.relu(
            jnp.dot(wt_ref[...], eht, preferred_element_type=jnp.float32)
            + b_ref[...]
        )
        hxt = hx_ref[...].T
        p = tit.reshape(-1, hdim, tile) * hxt[None, :, :]
        m = p.sum(axis=1)
        out_ref[...] = jnp.where(valid, m, 0.0).T

    msgs(wit_ref, bi_ref, hd_ref, mi_ref)
    msgs(wot_ref, bo_ref, hs_ref, mo_ref)


def _edge_layer(e_true, base, edge_inp_t_seg, w_edge, b_edge, w_mi, b_mi,
                w_mo, b_mo, hd, hs, tile=2048):
    e_pad = edge_inp_t_seg.shape[1]
    h = w_edge.shape[1]
    h2 = w_mi.shape[1]
    grid = (e_pad // tile,)
    const = lambda i: (0, 0)
    return pl.pallas_call(
        functools.partial(_edge_layer_body, e_true, base, tile, h),
        grid=grid,
        in_specs=[
            pl.BlockSpec((2, tile), lambda i: (0, i)),
            pl.BlockSpec((h, 2), const),
            pl.BlockSpec((h, 1), const),
            pl.BlockSpec((h2, h), const),
            pl.BlockSpec((h2, 1), const),
            pl.BlockSpec((h2, h), const),
            pl.BlockSpec((h2, 1), const),
            pl.BlockSpec((tile, h), lambda i: (i, 0)),
            pl.BlockSpec((tile, h), lambda i: (i, 0)),
        ],
        out_specs=[
            pl.BlockSpec((tile, h // 2), lambda i: (i, 0)),
            pl.BlockSpec((tile, h // 2), lambda i: (i, 0)),
        ],
        out_shape=[
            jax.ShapeDtypeStruct((e_pad, h // 2), jnp.float32),
            jax.ShapeDtypeStruct((e_pad, h // 2), jnp.float32),
        ],
    )(edge_inp_t_seg, w_edge.T, b_edge.reshape(h, 1), w_mi.T,
      b_mi.reshape(h2, 1), w_mo.T, b_mo.reshape(h2, 1), hd, hs)


def _update_body(h_ref, *refs):
    o_ref = refs[-1]
    half = (len(refs) - 1) // 2
    mi = sum(r[0] + r[1] for r in refs[:half])
    mo = sum(r[0] + r[1] for r in refs[half:-1])
    o_ref[...] = jax.nn.relu(h_ref[...] + jnp.concatenate([mi, mo], axis=1))


def _node_update(h, mi_ps, mo_ps):
    return pl.pallas_call(
        _update_body,
        out_shape=jax.ShapeDtypeStruct(h.shape, jnp.float32),
    )(h, *mi_ps, *mo_ps)


def _readout_body(nseg, h_ref, *refs):
    (wg_ref, bg_ref, wf_ref, bf_ref, w1_ref, b1_ref, w2_ref, b2_ref,
     o_ref) = refs[2 * nseg:]
    mi = sum(r[0] + r[1] for r in refs[:nseg])
    mo = sum(r[0] + r[1] for r in refs[nseg:2 * nseg])
    h = jax.nn.relu(h_ref[...] + jnp.concatenate([mi, mo], axis=1))
    g = jnp.dot(h, wg_ref[...], preferred_element_type=jnp.float32) + bg_ref[...]
    p = jnp.exp(g - jnp.max(g))
    z = jnp.sum(p)
    feat = jnp.dot(h, wf_ref[...], preferred_element_type=jnp.float32) + bf_ref[...]
    r = jnp.sum(p * feat, axis=0, keepdims=True) / z
    o1 = jax.nn.relu(
        jnp.dot(r, w1_ref[...], preferred_element_type=jnp.float32) + b1_ref[...]
    )
    o_ref[...] = jnp.dot(o1, w2_ref[...], preferred_element_type=jnp.float32) + b2_ref[...]


def _readout(h, mi_ps, mo_ps, w_gate, b_gate, w_feat, b_feat, w_f1, b_f1,
             w_f2, b_f2):
    hdim = h.shape[1]
    out = pl.pallas_call(
        functools.partial(_readout_body, len(mi_ps)),
        out_shape=jax.ShapeDtypeStruct((1, 1), jnp.float32),
    )(h, *mi_ps, *mo_ps, w_gate, b_gate.reshape(1, 1), w_feat,
      b_feat.reshape(1, hdim), w_f1, b_f1.reshape(1, hdim), w_f2,
      b_f2.reshape(1, 1))
    return out.reshape(())


def _gather_body(chunk, nidx, h_hbm, dst_hbm, src_hbm, hd_hbm, hs_hbm,
                 idx_v, rows_v, sem):
    c = lax.axis_index("c")
    s = lax.axis_index("s")
    wid = c * _NS + s
    base = wid * chunk

    def gather_one(idx3, out_hbm):
        pltpu.sync_copy(idx3.at[wid], idx_v)
        descs = []
        for j in range(nidx):
            descs.append(pltpu.async_copy(
                h_hbm.at[idx_v.at[j]],
                rows_v.at[pl.ds(j * _IDX_CHUNK, _IDX_CHUNK)],
                sem,
            ))
        for d in descs:
            d.wait()
        pltpu.sync_copy(rows_v, out_hbm.at[pl.ds(base, chunk)])

    gather_one(dst_hbm, hd_hbm)
    gather_one(src_hbm, hs_hbm)


def _sc_gather(h, dst3, src3):
    n, hdim = h.shape
    nw, nidx, _ = dst3.shape
    chunk = nidx * _IDX_CHUNK
    e_pad = nw * chunk
    mesh = plsc.VectorSubcoreMesh(core_axis_name="c", subcore_axis_name="s")
    kfn = pl.kernel(
        functools.partial(_gather_body, chunk, nidx),
        out_type=[
            jax.ShapeDtypeStruct((e_pad, hdim), jnp.float32),
            jax.ShapeDtypeStruct((e_pad, hdim), jnp.float32),
        ],
        mesh=mesh,
        scratch_types=[
            pltpu.VMEM((nidx, _IDX_CHUNK), jnp.int32),
            pltpu.VMEM((chunk, hdim), jnp.float32),
            pltpu.SemaphoreType.DMA,
        ],
        compiler_params=pltpu.CompilerParams(use_tc_tiling_on_sc=False),
    )
    return kfn(h, dst3, src3)


def _scatter_body(nidx, mi3_hbm, mo3_hbm, dst_hbm, src_hbm, zero_hbm,
                  mip_hbm, mop_hbm, msg_v, idx_v, acc_i, acc_o):
    c = lax.axis_index("c")
    s = lax.axis_index("s")
    wid = c * _NS + s

    @pl.when(s == 0)
    def _():
        pltpu.sync_copy(zero_hbm, acc_i)

    @pl.when(s == 1)
    def _():
        pltpu.sync_copy(zero_hbm, acc_o)

    plsc.subcore_barrier()

    def scatter_one(msg3, idx3, acc):
        pltpu.sync_copy(msg3.at[wid], msg_v)
        pltpu.sync_copy(idx3.at[wid], idx_v)
        for j in range(nidx):
            pltpu.sync_copy(
                msg_v.at[pl.ds(j * _IDX_CHUNK, _IDX_CHUNK)],
                acc.at[idx_v.at[j]],
                add=True,
            )

    scatter_one(mi3_hbm, dst_hbm, acc_i)
    scatter_one(mo3_hbm, src_hbm, acc_o)

    plsc.subcore_barrier()

    @pl.when(s == 0)
    def _():
        pltpu.sync_copy(acc_i, mip_hbm.at[c])

    @pl.when(s == 1)
    def _():
        pltpu.sync_copy(acc_o, mop_hbm.at[c])


def _sc_scatter(mi3, mo3, dst3, src3, zeros_nk):
    nw, chunk, k = mi3.shape
    nidx = chunk // _IDX_CHUNK
    n = zeros_nk.shape[0]
    mesh = plsc.VectorSubcoreMesh(core_axis_name="c", subcore_axis_name="s")
    kfn = pl.kernel(
        functools.partial(_scatter_body, nidx),
        out_type=[
            jax.ShapeDtypeStruct((_NC, n, k), jnp.float32),
            jax.ShapeDtypeStruct((_NC, n, k), jnp.float32),
        ],
        mesh=mesh,
        scratch_types=[
            pltpu.VMEM((chunk, k), jnp.float32),
            pltpu.VMEM((nidx, _IDX_CHUNK), jnp.int32),
            pltpu.VMEM_SHARED((n, k), jnp.float32),
            pltpu.VMEM_SHARED((n, k), jnp.float32),
        ],
        compiler_params=pltpu.CompilerParams(use_tc_tiling_on_sc=False),
    )
    return kfn(mi3, mo3, dst3, src3, zeros_nk)


def kernel(node_inp, edge_inp, W_node, b_node, W_edge, b_edge, W_mi, b_mi,
           W_mo, b_mo, W_gate, b_gate, W_feat, b_feat, W_f1, b_f1, W_f2,
           b_f2, edge_index):
    n = node_inp.shape[0]
    e = edge_inp.shape[0]
    hdim = W_node.shape[1]
    nlayers = W_mi.shape[0]
    nseg = 2

    unit = nseg * _NW * _IDX_CHUNK
    e_pad = ((e + unit - 1) // unit) * unit
    seg = e_pad // nseg
    chunk = seg // _NW
    nidx = chunk // _IDX_CHUNK

    pad = e_pad - e
    src = jnp.pad(edge_index[0], (0, pad))
    dst = jnp.pad(edge_index[1], (0, pad))
    dst4 = dst.reshape(nseg, _NW, nidx, _IDX_CHUNK)
    src4 = src.reshape(nseg, _NW, nidx, _IDX_CHUNK)
    edge_inp_t_pad = jnp.pad(edge_inp, ((0, pad), (0, 0))).T
    zeros_nk = jnp.zeros((n, hdim // 2), jnp.float32)

    h = _node_encoder(node_inp, W_node, b_node)

    mi_ps = mo_ps = None
    for l in range(nlayers):
        if l > 0:
            h = _node_update(h, mi_ps, mo_ps)
        gathered = [_sc_gather(h, dst4[s], src4[s]) for s in range(nseg)]
        msgs = [
            _edge_layer(
                e, s * seg, lax.slice(edge_inp_t_pad, (0, s * seg),
                                      (2, (s + 1) * seg)),
                W_edge, b_edge, W_mi[l], b_mi[l], W_mo[l], b_mo[l],
                gathered[s][0], gathered[s][1])
            for s in range(nseg)
        ]
        parts = [
            _sc_scatter(
                msgs[s][0].reshape(_NW, chunk, hdim // 2),
                msgs[s][1].reshape(_NW, chunk, hdim // 2),
                dst4[s], src4[s], zeros_nk)
            for s in range(nseg)
        ]
        mi_ps = [p[0] for p in parts]
        mo_ps = [p[1] for p in parts]

    return _readout(h, mi_ps, mo_ps, W_gate, b_gate, W_feat, b_feat,
                    W_f1, b_f1, W_f2, b_f2)
